# baseline (speedup 1.0000x reference)
import functools

import jax
import jax.numpy as jnp
from jax import lax
from jax.experimental import pallas as pl
from jax.experimental.pallas import tpu as pltpu
from jax.experimental.pallas import tpu_sc as plsc

_VOCAB_ROWS = 1000001
_D = 32
_B = 16384
_MAX_VOCAB = 1000000.0
_ALPHA = 0.75

_NC = 2
_NS = 16
_NW = _NC * _NS
_BPW = _B // _NW
_CHUNK = 128
_NCHUNK = _BPW // _CHUNK
_GROUPS = _BPW // 16


def _sc_body(t_ind, c_ind, t_emb, c_emb, t_bias, c_bias, out_hbm,
             tidx_v, cidx_v, trows_v, crows_v, tb_v, cb_v, s_v, sem):
    wid = lax.axis_index("s") * _NC + lax.axis_index("c")
    base = wid * _BPW

    pltpu.sync_copy(t_ind.at[wid], tidx_v)
    pltpu.sync_copy(c_ind.at[wid], cidx_v)

    copies = []
    for j in range(_NCHUNK):
        r = pl.ds(j * _CHUNK, _CHUNK)
        copies.append(pltpu.async_copy(t_emb.at[tidx_v.at[j]], trows_v.at[r], sem))
        copies.append(pltpu.async_copy(c_emb.at[cidx_v.at[j]], crows_v.at[r], sem))
        copies.append(pltpu.async_copy(t_bias.at[tidx_v.at[j]], tb_v.at[r], sem))
        copies.append(pltpu.async_copy(c_bias.at[cidx_v.at[j]], cb_v.at[r], sem))
    for c in copies:
        c.wait()

    lane = lax.iota(jnp.int32, 16)

    def group(g, carry):
        rows = g * 16 + lane
        acc = tb_v[pl.ds(g * 16, 16)] + cb_v[pl.ds(g * 16, 16)]
        for d in range(_D):
            col = jnp.full((16,), d, jnp.int32)
            tv = plsc.load_gather(trows_v, [rows, col])
            cv = plsc.load_gather(crows_v, [rows, col])
            acc = acc + tv * cv
        s_v[pl.ds(g * 16, 16)] = acc
        return carry

    lax.fori_loop(0, _GROUPS, group, 0, unroll=False)

    pltpu.sync_copy(s_v, out_hbm.at[pl.ds(base, _BPW)])


@functools.cache
def _make_sc_dot():
    @functools.partial(
        pl.kernel,
        mesh=plsc.VectorSubcoreMesh(core_axis_name="c", subcore_axis_name="s"),
        out_type=jax.ShapeDtypeStruct((_B,), jnp.float32),
        compiler_params=pltpu.CompilerParams(
            needs_layout_passes=False, use_tc_tiling_on_sc=False),
        scratch_types=[
            pltpu.VMEM((_NCHUNK, _CHUNK), jnp.int32),
            pltpu.VMEM((_NCHUNK, _CHUNK), jnp.int32),
            pltpu.VMEM((_BPW, _D), jnp.float32),
            pltpu.VMEM((_BPW, _D), jnp.float32),
            pltpu.VMEM((_BPW,), jnp.float32),
            pltpu.VMEM((_BPW,), jnp.float32),
            pltpu.VMEM((_BPW,), jnp.float32),
            pltpu.SemaphoreType.DMA,
        ],
    )
    def _sc_dot(t_ind, c_ind, t_emb, c_emb, t_bias, c_bias, out_hbm, *scratch):
        _sc_body(t_ind, c_ind, t_emb, c_emb, t_bias, c_bias, out_hbm, *scratch)

    return _sc_dot


def _tc_tail_body(s_ref, co_ref, out_ref):
    s = s_ref[...]
    co = co_ref[...]
    w = jnp.minimum(1.0, jnp.power(co * (1.0 / _MAX_VOCAB), _ALPHA))
    diff = s - jnp.log(co + 1.0)
    out_ref[0, 0] = jnp.sum(w * diff * diff)


_tc_tail = pl.pallas_call(
    _tc_tail_body,
    out_shape=jax.ShapeDtypeStruct((1, 1), jnp.float32),
    out_specs=pl.BlockSpec(memory_space=pltpu.SMEM),
)


def kernel(target_ind, context_ind, co_occurs, target_embeddings,
           context_embeddings, target_biases, context_biases):
    tind = target_ind.astype(jnp.int32).reshape(_NW, _NCHUNK, _CHUNK)
    cind = context_ind.astype(jnp.int32).reshape(_NW, _NCHUNK, _CHUNK)
    s = _make_sc_dot()(tind, cind, target_embeddings, context_embeddings,
                       target_biases, context_biases)
    cost = _tc_tail(s.reshape(128, 128), co_occurs.astype(jnp.float32).reshape(128, 128))
    return cost[0, 0]
